# Initial kernel scaffold; baseline (speedup 1.0000x reference)
import jax
import jax.numpy as jnp
from jax import lax
from jax.experimental import pallas as pl
from jax.experimental.pallas import tpu as pltpu
from jax.experimental.pallas import tpu_sc as plsc

BATCH = 16384
HIST = 50
DIM = 50
NC = 2
NS = 16
NW = NC * NS
ROWS_W = BATCH // NW
IDX_W = ROWS_W * HIST
IT = 80
GPC = 8
CR = GPC * HIST
JPC = CR // IT
NCHUNK = IDX_W // CR
XROWS_W = IDX_W // IT

_mesh = plsc.VectorSubcoreMesh(core_axis_name="c", subcore_axis_name="s")


def _sc_body(xr_hbm, xw_hbm, etd_hbm, ewae_hbm, out_hbm, idx_v, rows_v,
             acc_v, sem):
    wid = lax.axis_index("s") * NC + lax.axis_index("c")
    neg = jnp.full((16,), -3.0e38, jnp.float32)
    for o in range(0, 128, 16):
        acc_v[pl.ds(o, 16)] = neg

    def pool_table(x_hbm, tbl_hbm, aoff):
        xbase = wid * XROWS_W

        def chunk_body(c, carry):
            pltpu.sync_copy(x_hbm.at[pl.ds(xbase + c * JPC, JPC)], idx_v)
            cps = [
                pltpu.async_copy(
                    tbl_hbm.at[idx_v.at[j]], rows_v.at[pl.ds(j * IT, IT)],
                    sem)
                for j in range(JPC)
            ]
            for cp in cps:
                cp.wait()
            zero = jnp.zeros((16,), jnp.float32)
            for g in range(GPC):
                gbase = g * HIST

                def hbody(i, acc):
                    a0, a1, a2, a3 = acc
                    for u in range(5):
                        r = gbase + i * 5 + u
                        a0 = a0 + rows_v[r, pl.ds(0, 16)]
                        a1 = a1 + rows_v[r, pl.ds(16, 16)]
                        a2 = a2 + rows_v[r, pl.ds(32, 16)]
                        a3 = a3 + rows_v[r, pl.ds(34, 16)]
                    return (a0, a1, a2, a3)

                a0, a1, a2, a3 = lax.fori_loop(
                    0, HIST // 5, hbody, (zero, zero, zero, zero))
                for o, a in ((0, a0), (16, a1), (32, a2), (48, a3)):
                    acc_v[pl.ds(aoff + o, 16)] = jnp.maximum(
                        acc_v[pl.ds(aoff + o, 16)], a)
            return carry

        lax.fori_loop(0, NCHUNK, chunk_body, 0)

    pool_table(xr_hbm, etd_hbm, 0)
    pool_table(xw_hbm, ewae_hbm, 64)
    pltpu.sync_copy(acc_v, out_hbm.at[wid])


_sc_pool = pl.kernel(
    _sc_body,
    out_type=jax.ShapeDtypeStruct((NW, 128), jnp.float32),
    mesh=_mesh,
    scratch_types=[
        pltpu.VMEM((JPC, IT), jnp.int32),
        pltpu.VMEM((CR, DIM), jnp.float32),
        pltpu.VMEM((128,), jnp.float32),
        pltpu.SemaphoreType.DMA,
    ],
)


def _head_body(p_ref, w_ref, b_ref, y_ref, pred_ref, loss_ref):
    m = jnp.max(p_ref[...], axis=0, keepdims=True)
    hid = jnp.concatenate(
        [m[:, 0:48], m[:, 62:64], m[:, 64:112], m[:, 126:128]],
        axis=1) * (1.0 / HIST)
    logits = lax.dot_general(
        hid, w_ref[...], (((1,), (1,)), ((), ()))) + b_ref[...]
    ex = jnp.exp(logits - jnp.max(logits, axis=1, keepdims=True))
    pred = ex / jnp.sum(ex, axis=1, keepdims=True)
    mx = jnp.max(pred, axis=1, keepdims=True)
    lse = jnp.log(jnp.sum(jnp.exp(pred - mx), axis=1, keepdims=True)) + mx
    onehot = (y_ref[...] == 1).astype(jnp.float32)
    picked = jnp.sum(pred * onehot, axis=1, keepdims=True)
    pred_ref[...] = pred
    loss_ref[...] = lse - picked


_head = pl.pallas_call(
    _head_body,
    out_shape=[jax.ShapeDtypeStruct((1, 4), jnp.float32),
               jax.ShapeDtypeStruct((1, 1), jnp.float32)],
)


def kernel(x_random, x_response, y, E_td, E_wae, w_cat, b_cat):
    xr = x_random.reshape(-1, IT)
    xw = x_response.reshape(-1, IT)
    partials = _sc_pool(xr, xw, E_td, E_wae)
    pred, loss = _head(partials, w_cat, b_cat.reshape(1, 4), y.reshape(1, 4))
    return pred.reshape(4), loss.reshape(())


# speedup vs baseline: 3.1815x; 3.1815x over previous
import jax
import jax.numpy as jnp
from jax import lax
from jax.experimental import pallas as pl
from jax.experimental.pallas import tpu as pltpu
from jax.experimental.pallas import tpu_sc as plsc

BATCH = 16384
HIST = 50
DIM = 50
PDIM = 56
NC = 2
NS = 16
NW = NC * NS
ROWS_W = BATCH // NW
GPC = 8
NCHUNK = ROWS_W // GPC
VTD = 1000000
VWAE = 100000

_mesh = plsc.VectorSubcoreMesh(core_axis_name="c", subcore_axis_name="s")


def _transpose_block(src_v, dst_v, nrows):
    i16 = lax.iota(jnp.int32, 16)

    def rbody(i, carry):
        for u in range(4):
            r = i * 4 + u
            rs = jnp.full((16,), r, jnp.int32)
            for d0 in (0, 16, 32, 34):
                piece = plsc.load_gather(src_v, [i16 + d0, rs])
                dst_v[pl.ds(r * PDIM + d0, 16)] = piece
        return carry

    lax.fori_loop(0, nrows // 4, rbody, 0)


def _tx_body(dtd_hbm, dwae_hbm, atd_hbm, awae_hbm, otd_hbm, owae_hbm,
             src_a, src_b, dst_a, dst_b, semr, semw):
    wid = lax.axis_index("s") * NC + lax.axis_index("c")

    def do_table(src_hbm, aux_hbm, out_hbm, nfull, tailw):
        nmain = (nfull // NW) & ~1
        npair = nmain // 2
        nextra = nfull - nmain * NW

        def rd(b, buf):
            return pltpu.async_copy(src_hbm.at[:, pl.ds(b * 128, 128)],
                                    buf, semr)

        def rd_wait(b, buf):
            pltpu.make_async_copy(src_hbm.at[:, pl.ds(b * 128, 128)],
                                  buf, semr).wait()

        def wr(b, buf):
            return pltpu.async_copy(
                buf, out_hbm.at[pl.ds(b * 128 * PDIM, 128 * PDIM)], semw)

        def wr_wait(b, buf):
            pltpu.make_async_copy(
                buf, out_hbm.at[pl.ds(b * 128 * PDIM, 128 * PDIM)],
                semw).wait()

        rd(wid, src_a)
        rd(wid + NW, src_b)

        def pair_body(k, carry):
            b_a = wid + (2 * k) * NW
            b_b = b_a + NW
            rd_wait(b_a, src_a)

            @pl.when(k > 0)
            def _():
                wr_wait(b_a - 2 * NW, dst_a)

            _transpose_block(src_a, dst_a, 128)
            wr(b_a, dst_a)

            @pl.when(k < npair - 1)
            def _():
                rd(b_a + 2 * NW, src_a)

            rd_wait(b_b, src_b)

            @pl.when(k > 0)
            def _():
                wr_wait(b_b - 2 * NW, dst_b)

            _transpose_block(src_b, dst_b, 128)
            wr(b_b, dst_b)

            @pl.when(k < npair - 1)
            def _():
                rd(b_b + 2 * NW, src_b)

            return carry

        lax.fori_loop(0, npair, pair_body, 0)
        wr_wait(wid + (nmain - 2) * NW, dst_a)
        wr_wait(wid + (nmain - 1) * NW, dst_b)

        @pl.when(wid < nextra)
        def _():
            b = nmain * NW + wid
            pltpu.sync_copy(src_hbm.at[:, pl.ds(b * 128, 128)], src_a)
            _transpose_block(src_a, dst_a, 128)
            pltpu.sync_copy(dst_a,
                            out_hbm.at[pl.ds(b * 128 * PDIM, 128 * PDIM)])

        @pl.when(wid == NW - 1)
        def _():
            pltpu.sync_copy(aux_hbm.at[:, :], src_a)
            _transpose_block(src_a, dst_a, tailw)
            pltpu.sync_copy(dst_a.at[pl.ds(0, tailw * PDIM)],
                            out_hbm.at[pl.ds(nfull * 128 * PDIM,
                                             tailw * PDIM)])

    do_table(dtd_hbm, atd_hbm, otd_hbm, VTD // 128, VTD % 128)
    do_table(dwae_hbm, awae_hbm, owae_hbm, VWAE // 128, VWAE % 128)


_sc_transpose = pl.kernel(
    _tx_body,
    out_type=(jax.ShapeDtypeStruct((VTD * PDIM,), jnp.float32),
              jax.ShapeDtypeStruct((VWAE * PDIM,), jnp.float32)),
    mesh=_mesh,
    scratch_types=[
        pltpu.VMEM((DIM, 128), jnp.float32),
        pltpu.VMEM((DIM, 128), jnp.float32),
        pltpu.VMEM((128 * PDIM,), jnp.float32),
        pltpu.VMEM((128 * PDIM,), jnp.float32),
        pltpu.SemaphoreType.DMA,
        pltpu.SemaphoreType.DMA,
    ],
    compiler_params=pltpu.CompilerParams(needs_layout_passes=False),
)


def _sc_body(xr_hbm, xw_hbm, etd_hbm, ewae_hbm, out_hbm, idx_a, idx_b,
             rows_a, rows_b, acc_v, semi, semg):
    wid = lax.axis_index("s") * NC + lax.axis_index("c")
    neg = jnp.full((16,), -3.0e38, jnp.float32)
    for rr in range(8):
        for o in range(0, 128, 16):
            acc_v[rr, pl.ds(o, 16)] = neg

    def pool_table(x_hbm, tbl_hbm, aoff):
        nbase = wid * ROWS_W

        def idx_rd(c, buf):
            return pltpu.async_copy(
                x_hbm.at[pl.ds(nbase + c * GPC, GPC)], buf, semi)

        def idx_wait(c, buf):
            pltpu.make_async_copy(
                x_hbm.at[pl.ds(nbase + c * GPC, GPC)], buf, semi).wait()

        def gat(ibuf, rbuf):
            for j in range(GPC):
                pltpu.async_copy(tbl_hbm.at[ibuf.at[j]], rbuf.at[j], semg)

        def gat_wait(ibuf, rbuf):
            for j in range(GPC):
                pltpu.make_async_copy(tbl_hbm.at[ibuf.at[j]], rbuf.at[j],
                                      semg).wait()

        def accum(rbuf):
            zero = jnp.zeros((16,), jnp.float32)
            for g in range(GPC):

                def hbody(i, acc):
                    a0, a1, a2, a3 = acc
                    for u in range(5):
                        r = i * 5 + u
                        a0 = a0 + rbuf[g, r, pl.ds(0, 16)]
                        a1 = a1 + rbuf[g, r, pl.ds(16, 16)]
                        a2 = a2 + rbuf[g, r, pl.ds(32, 16)]
                        a3 = a3 + rbuf[g, r, pl.ds(34, 16)]
                    return (a0, a1, a2, a3)

                a0, a1, a2, a3 = lax.fori_loop(
                    0, HIST // 5, hbody, (zero, zero, zero, zero))
                for o, a in ((0, a0), (16, a1), (32, a2), (48, a3)):
                    acc_v[0, pl.ds(aoff + o, 16)] = jnp.maximum(
                        acc_v[0, pl.ds(aoff + o, 16)], a)

        npair = NCHUNK // 2
        idx_rd(0, idx_a)
        idx_rd(1, idx_b)
        idx_wait(0, idx_a)
        gat(idx_a, rows_a)

        def pair_body(k, carry):
            c_a = 2 * k
            gat_wait(idx_a, rows_a)
            idx_wait(c_a + 1, idx_b)
            gat(idx_b, rows_b)

            @pl.when(k < npair - 1)
            def _():
                idx_rd(c_a + 2, idx_a)

            accum(rows_a)
            gat_wait(idx_b, rows_b)

            @pl.when(k < npair - 1)
            def _():
                idx_wait(c_a + 2, idx_a)
                gat(idx_a, rows_a)
                idx_rd(c_a + 3, idx_b)

            accum(rows_b)
            return carry

        lax.fori_loop(0, npair, pair_body, 0)

    pool_table(xr_hbm, etd_hbm, 0)
    pool_table(xw_hbm, ewae_hbm, 64)
    pltpu.sync_copy(acc_v, out_hbm.at[pl.ds(wid * 8, 8)])


_sc_pool = pl.kernel(
    _sc_body,
    out_type=jax.ShapeDtypeStruct((NW * 8, 128), jnp.float32),
    mesh=_mesh,
    scratch_types=[
        pltpu.VMEM((GPC, HIST), jnp.int32),
        pltpu.VMEM((GPC, HIST), jnp.int32),
        pltpu.VMEM((GPC, HIST, PDIM), jnp.float32),
        pltpu.VMEM((GPC, HIST, PDIM), jnp.float32),
        pltpu.VMEM((8, 128), jnp.float32),
        pltpu.SemaphoreType.DMA,
        pltpu.SemaphoreType.DMA,
    ],
    compiler_params=pltpu.CompilerParams(use_tc_tiling_on_sc=False),
)


def _head_body(p_ref, w_ref, b_ref, y_ref, pred_ref, loss_ref):
    m = jnp.max(p_ref[...], axis=0, keepdims=True)
    hid = jnp.concatenate(
        [m[:, 0:48], m[:, 62:64], m[:, 64:112], m[:, 126:128]],
        axis=1) * (1.0 / HIST)
    logits = lax.dot_general(
        hid, w_ref[...], (((1,), (1,)), ((), ()))) + b_ref[...]
    ex = jnp.exp(logits - jnp.max(logits, axis=1, keepdims=True))
    pred = ex / jnp.sum(ex, axis=1, keepdims=True)
    mx = jnp.max(pred, axis=1, keepdims=True)
    lse = jnp.log(jnp.sum(jnp.exp(pred - mx), axis=1, keepdims=True)) + mx
    onehot = (y_ref[...] == 1).astype(jnp.float32)
    picked = jnp.sum(pred * onehot, axis=1, keepdims=True)
    pred_ref[...] = pred
    loss_ref[...] = lse - picked


_head = pl.pallas_call(
    _head_body,
    out_shape=[jax.ShapeDtypeStruct((1, 4), jnp.float32),
               jax.ShapeDtypeStruct((1, 1), jnp.float32)],
)


def kernel(x_random, x_response, y, E_td, E_wae, w_cat, b_cat):
    aux_td = jnp.pad(E_td[VTD - VTD % 128:].T, ((0, 0), (0, 128 - VTD % 128)))
    aux_wae = jnp.pad(E_wae[VWAE - VWAE % 128:].T,
                      ((0, 0), (0, 128 - VWAE % 128)))
    otd, owae = _sc_transpose(E_td.T, E_wae.T, aux_td, aux_wae)
    partials = _sc_pool(x_random, x_response,
                        otd.reshape(VTD, PDIM), owae.reshape(VWAE, PDIM))
    pred, loss = _head(partials, w_cat, b_cat.reshape(1, 4), y.reshape(1, 4))
    return pred.reshape(4), loss.reshape(())


# speedup vs baseline: 3.1832x; 1.0006x over previous
import jax
import jax.numpy as jnp
from jax import lax
from jax.experimental import pallas as pl
from jax.experimental.pallas import tpu as pltpu
from jax.experimental.pallas import tpu_sc as plsc

BATCH = 16384
HIST = 50
DIM = 50
PDIM = 56
NC = 2
NS = 16
NW = NC * NS
ROWS_W = BATCH // NW
GPC = 8
NCHUNK = ROWS_W // GPC
VTD = 1000000
VWAE = 100000

_mesh = plsc.VectorSubcoreMesh(core_axis_name="c", subcore_axis_name="s")


SRCW = 129


def _transpose_block(src_v, dst_v, nrows):
    i16 = lax.iota(jnp.int32, 16)

    def rbody(i, carry):
        for u in range(4):
            r = i * 4 + u
            rs = jnp.full((16,), r, jnp.int32)
            for d0 in (0, 16, 32, 34):
                piece = plsc.load_gather(src_v, [i16 + d0, rs])
                dst_v[pl.ds(r * PDIM + d0, 16)] = piece
        return carry

    lax.fori_loop(0, nrows // 4, rbody, 0)


def _tx_body(dtd_hbm, dwae_hbm, atd_hbm, awae_hbm, otd_hbm, owae_hbm,
             src_a, src_b, dst_a, dst_b, semr, semw):
    wid = lax.axis_index("s") * NC + lax.axis_index("c")

    def do_table(src_hbm, aux_hbm, out_hbm, nfull, tailw):
        nmain = (nfull // NW) & ~1
        npair = nmain // 2
        nextra = nfull - nmain * NW

        def rd(b, buf):
            return pltpu.async_copy(src_hbm.at[:, pl.ds(b * 128, 128)],
                                    buf.at[:, pl.ds(0, 128)], semr)

        def rd_wait(b, buf):
            pltpu.make_async_copy(src_hbm.at[:, pl.ds(b * 128, 128)],
                                  buf.at[:, pl.ds(0, 128)], semr).wait()

        def wr(b, buf):
            return pltpu.async_copy(
                buf, out_hbm.at[pl.ds(b * 128 * PDIM, 128 * PDIM)], semw)

        def wr_wait(b, buf):
            pltpu.make_async_copy(
                buf, out_hbm.at[pl.ds(b * 128 * PDIM, 128 * PDIM)],
                semw).wait()

        rd(wid, src_a)
        rd(wid + NW, src_b)

        def pair_body(k, carry):
            b_a = wid + (2 * k) * NW
            b_b = b_a + NW
            rd_wait(b_a, src_a)

            @pl.when(k > 0)
            def _():
                wr_wait(b_a - 2 * NW, dst_a)

            _transpose_block(src_a, dst_a, 128)
            wr(b_a, dst_a)

            @pl.when(k < npair - 1)
            def _():
                rd(b_a + 2 * NW, src_a)

            rd_wait(b_b, src_b)

            @pl.when(k > 0)
            def _():
                wr_wait(b_b - 2 * NW, dst_b)

            _transpose_block(src_b, dst_b, 128)
            wr(b_b, dst_b)

            @pl.when(k < npair - 1)
            def _():
                rd(b_b + 2 * NW, src_b)

            return carry

        lax.fori_loop(0, npair, pair_body, 0)
        wr_wait(wid + (nmain - 2) * NW, dst_a)
        wr_wait(wid + (nmain - 1) * NW, dst_b)

        @pl.when(wid < nextra)
        def _():
            b = nmain * NW + wid
            pltpu.sync_copy(src_hbm.at[:, pl.ds(b * 128, 128)],
                            src_a.at[:, pl.ds(0, 128)])
            _transpose_block(src_a, dst_a, 128)
            pltpu.sync_copy(dst_a,
                            out_hbm.at[pl.ds(b * 128 * PDIM, 128 * PDIM)])

        @pl.when(wid == NW - 1)
        def _():
            pltpu.sync_copy(aux_hbm.at[:, :], src_a.at[:, pl.ds(0, 128)])
            _transpose_block(src_a, dst_a, tailw)
            pltpu.sync_copy(dst_a.at[pl.ds(0, tailw * PDIM)],
                            out_hbm.at[pl.ds(nfull * 128 * PDIM,
                                             tailw * PDIM)])

    do_table(dtd_hbm, atd_hbm, otd_hbm, VTD // 128, VTD % 128)
    do_table(dwae_hbm, awae_hbm, owae_hbm, VWAE // 128, VWAE % 128)


_sc_transpose = pl.kernel(
    _tx_body,
    out_type=(jax.ShapeDtypeStruct((VTD * PDIM,), jnp.float32),
              jax.ShapeDtypeStruct((VWAE * PDIM,), jnp.float32)),
    mesh=_mesh,
    scratch_types=[
        pltpu.VMEM((DIM, SRCW), jnp.float32),
        pltpu.VMEM((DIM, SRCW), jnp.float32),
        pltpu.VMEM((128 * PDIM,), jnp.float32),
        pltpu.VMEM((128 * PDIM,), jnp.float32),
        pltpu.SemaphoreType.DMA,
        pltpu.SemaphoreType.DMA,
    ],
    compiler_params=pltpu.CompilerParams(needs_layout_passes=False),
)


def _sc_body(xr_hbm, xw_hbm, etd_hbm, ewae_hbm, out_hbm, idx_a, idx_b,
             rows_a, rows_b, acc_v, semi, semg):
    wid = lax.axis_index("s") * NC + lax.axis_index("c")
    neg = jnp.full((16,), -3.0e38, jnp.float32)
    for rr in range(8):
        for o in range(0, 128, 16):
            acc_v[rr, pl.ds(o, 16)] = neg

    def pool_table(x_hbm, tbl_hbm, aoff):
        nbase = wid * ROWS_W

        def idx_rd(c, buf):
            return pltpu.async_copy(
                x_hbm.at[pl.ds(nbase + c * GPC, GPC)], buf, semi)

        def idx_wait(c, buf):
            pltpu.make_async_copy(
                x_hbm.at[pl.ds(nbase + c * GPC, GPC)], buf, semi).wait()

        def gat(ibuf, rbuf):
            for j in range(GPC):
                pltpu.async_copy(tbl_hbm.at[ibuf.at[j]], rbuf.at[j], semg)

        def gat_wait(ibuf, rbuf):
            for j in range(GPC):
                pltpu.make_async_copy(tbl_hbm.at[ibuf.at[j]], rbuf.at[j],
                                      semg).wait()

        def accum(rbuf):
            zero = jnp.zeros((16,), jnp.float32)
            for g in range(GPC):

                def hbody(i, acc):
                    a0, a1, a2, a3 = acc
                    for u in range(5):
                        r = i * 5 + u
                        a0 = a0 + rbuf[g, r, pl.ds(0, 16)]
                        a1 = a1 + rbuf[g, r, pl.ds(16, 16)]
                        a2 = a2 + rbuf[g, r, pl.ds(32, 16)]
                        a3 = a3 + rbuf[g, r, pl.ds(34, 16)]
                    return (a0, a1, a2, a3)

                a0, a1, a2, a3 = lax.fori_loop(
                    0, HIST // 5, hbody, (zero, zero, zero, zero))
                for o, a in ((0, a0), (16, a1), (32, a2), (48, a3)):
                    acc_v[0, pl.ds(aoff + o, 16)] = jnp.maximum(
                        acc_v[0, pl.ds(aoff + o, 16)], a)

        npair = NCHUNK // 2
        idx_rd(0, idx_a)
        idx_rd(1, idx_b)
        idx_wait(0, idx_a)
        gat(idx_a, rows_a)

        def pair_body(k, carry):
            c_a = 2 * k
            gat_wait(idx_a, rows_a)
            idx_wait(c_a + 1, idx_b)
            gat(idx_b, rows_b)

            @pl.when(k < npair - 1)
            def _():
                idx_rd(c_a + 2, idx_a)

            accum(rows_a)
            gat_wait(idx_b, rows_b)

            @pl.when(k < npair - 1)
            def _():
                idx_wait(c_a + 2, idx_a)
                gat(idx_a, rows_a)
                idx_rd(c_a + 3, idx_b)

            accum(rows_b)
            return carry

        lax.fori_loop(0, npair, pair_body, 0)

    pool_table(xr_hbm, etd_hbm, 0)
    pool_table(xw_hbm, ewae_hbm, 64)
    pltpu.sync_copy(acc_v, out_hbm.at[pl.ds(wid * 8, 8)])


_sc_pool = pl.kernel(
    _sc_body,
    out_type=jax.ShapeDtypeStruct((NW * 8, 128), jnp.float32),
    mesh=_mesh,
    scratch_types=[
        pltpu.VMEM((GPC, HIST), jnp.int32),
        pltpu.VMEM((GPC, HIST), jnp.int32),
        pltpu.VMEM((GPC, HIST, PDIM), jnp.float32),
        pltpu.VMEM((GPC, HIST, PDIM), jnp.float32),
        pltpu.VMEM((8, 128), jnp.float32),
        pltpu.SemaphoreType.DMA,
        pltpu.SemaphoreType.DMA,
    ],
    compiler_params=pltpu.CompilerParams(use_tc_tiling_on_sc=False),
)


def _head_body(p_ref, w_ref, b_ref, y_ref, pred_ref, loss_ref):
    m = jnp.max(p_ref[...], axis=0, keepdims=True)
    hid = jnp.concatenate(
        [m[:, 0:48], m[:, 62:64], m[:, 64:112], m[:, 126:128]],
        axis=1) * (1.0 / HIST)
    logits = lax.dot_general(
        hid, w_ref[...], (((1,), (1,)), ((), ()))) + b_ref[...]
    ex = jnp.exp(logits - jnp.max(logits, axis=1, keepdims=True))
    pred = ex / jnp.sum(ex, axis=1, keepdims=True)
    mx = jnp.max(pred, axis=1, keepdims=True)
    lse = jnp.log(jnp.sum(jnp.exp(pred - mx), axis=1, keepdims=True)) + mx
    onehot = (y_ref[...] == 1).astype(jnp.float32)
    picked = jnp.sum(pred * onehot, axis=1, keepdims=True)
    pred_ref[...] = pred
    loss_ref[...] = lse - picked


_head = pl.pallas_call(
    _head_body,
    out_shape=[jax.ShapeDtypeStruct((1, 4), jnp.float32),
               jax.ShapeDtypeStruct((1, 1), jnp.float32)],
)


def kernel(x_random, x_response, y, E_td, E_wae, w_cat, b_cat):
    aux_td = jnp.pad(E_td[VTD - VTD % 128:].T, ((0, 0), (0, 128 - VTD % 128)))
    aux_wae = jnp.pad(E_wae[VWAE - VWAE % 128:].T,
                      ((0, 0), (0, 128 - VWAE % 128)))
    otd, owae = _sc_transpose(E_td.T, E_wae.T, aux_td, aux_wae)
    partials = _sc_pool(x_random, x_response,
                        otd.reshape(VTD, PDIM), owae.reshape(VWAE, PDIM))
    pred, loss = _head(partials, w_cat, b_cat.reshape(1, 4), y.reshape(1, 4))
    return pred.reshape(4), loss.reshape(())


# speedup vs baseline: 5.3947x; 1.6947x over previous
import jax
import jax.numpy as jnp
from jax import lax
from jax.experimental import pallas as pl
from jax.experimental.pallas import tpu as pltpu
from jax.experimental.pallas import tpu_sc as plsc

BATCH = 16384
HIST = 50
DIM = 50
PDIM = 56
NC = 2
NS = 16
NW = NC * NS
ROWS_W = BATCH // NW
GPC = 8
NCHUNK = ROWS_W // GPC
VTD = 1000000
VWAE = 100000

_mesh = plsc.VectorSubcoreMesh(core_axis_name="c", subcore_axis_name="s")


SRCW = 129


def _transpose_block(src_v, dst_v, nrows):
    i16 = lax.iota(jnp.int32, 16)

    @plsc.parallel_loop(0, nrows, 1, unroll=4)
    def _(r):
        rs = jnp.full((16,), r, jnp.int32)
        for d0 in (0, 16, 32, 34):
            piece = plsc.load_gather(src_v, [i16 + d0, rs])
            dst_v[pl.ds(r * PDIM + d0, 16)] = piece


def _tx_body(dtd_hbm, dwae_hbm, atd_hbm, awae_hbm, otd_hbm, owae_hbm,
             src_a, src_b, dst_a, dst_b, semr, semw):
    wid = lax.axis_index("s") * NC + lax.axis_index("c")

    def do_table(src_hbm, aux_hbm, out_hbm, nfull, tailw):
        nmain = (nfull // NW) & ~1
        npair = nmain // 2
        nextra = nfull - nmain * NW

        def rd(b, buf):
            return pltpu.async_copy(src_hbm.at[:, pl.ds(b * 128, 128)],
                                    buf.at[:, pl.ds(0, 128)], semr)

        def rd_wait(b, buf):
            pltpu.make_async_copy(src_hbm.at[:, pl.ds(b * 128, 128)],
                                  buf.at[:, pl.ds(0, 128)], semr).wait()

        def wr(b, buf):
            return pltpu.async_copy(
                buf, out_hbm.at[pl.ds(b * 128 * PDIM, 128 * PDIM)], semw)

        def wr_wait(b, buf):
            pltpu.make_async_copy(
                buf, out_hbm.at[pl.ds(b * 128 * PDIM, 128 * PDIM)],
                semw).wait()

        rd(wid, src_a)
        rd(wid + NW, src_b)

        def pair_body(k, carry):
            b_a = wid + (2 * k) * NW
            b_b = b_a + NW
            rd_wait(b_a, src_a)

            @pl.when(k > 0)
            def _():
                wr_wait(b_a - 2 * NW, dst_a)

            _transpose_block(src_a, dst_a, 128)
            wr(b_a, dst_a)

            @pl.when(k < npair - 1)
            def _():
                rd(b_a + 2 * NW, src_a)

            rd_wait(b_b, src_b)

            @pl.when(k > 0)
            def _():
                wr_wait(b_b - 2 * NW, dst_b)

            _transpose_block(src_b, dst_b, 128)
            wr(b_b, dst_b)

            @pl.when(k < npair - 1)
            def _():
                rd(b_b + 2 * NW, src_b)

            return carry

        lax.fori_loop(0, npair, pair_body, 0)
        wr_wait(wid + (nmain - 2) * NW, dst_a)
        wr_wait(wid + (nmain - 1) * NW, dst_b)

        @pl.when(wid < nextra)
        def _():
            b = nmain * NW + wid
            pltpu.sync_copy(src_hbm.at[:, pl.ds(b * 128, 128)],
                            src_a.at[:, pl.ds(0, 128)])
            _transpose_block(src_a, dst_a, 128)
            pltpu.sync_copy(dst_a,
                            out_hbm.at[pl.ds(b * 128 * PDIM, 128 * PDIM)])

        @pl.when(wid == NW - 1)
        def _():
            pltpu.sync_copy(aux_hbm.at[:, :], src_a.at[:, pl.ds(0, 128)])
            _transpose_block(src_a, dst_a, tailw)
            pltpu.sync_copy(dst_a.at[pl.ds(0, tailw * PDIM)],
                            out_hbm.at[pl.ds(nfull * 128 * PDIM,
                                             tailw * PDIM)])

    do_table(dtd_hbm, atd_hbm, otd_hbm, VTD // 128, VTD % 128)
    do_table(dwae_hbm, awae_hbm, owae_hbm, VWAE // 128, VWAE % 128)


_sc_transpose = pl.kernel(
    _tx_body,
    out_type=(jax.ShapeDtypeStruct((VTD * PDIM,), jnp.float32),
              jax.ShapeDtypeStruct((VWAE * PDIM,), jnp.float32)),
    mesh=_mesh,
    scratch_types=[
        pltpu.VMEM((DIM, SRCW), jnp.float32),
        pltpu.VMEM((DIM, SRCW), jnp.float32),
        pltpu.VMEM((128 * PDIM,), jnp.float32),
        pltpu.VMEM((128 * PDIM,), jnp.float32),
        pltpu.SemaphoreType.DMA,
        pltpu.SemaphoreType.DMA,
    ],
    compiler_params=pltpu.CompilerParams(needs_layout_passes=False),
)


def _sc_body(xr_hbm, xw_hbm, etd_hbm, ewae_hbm, out_hbm, idx_a, idx_b,
             rows_a, rows_b, acc_v, semi, semg):
    wid = lax.axis_index("s") * NC + lax.axis_index("c")
    neg = jnp.full((16,), -3.0e38, jnp.float32)
    for rr in range(8):
        for o in range(0, 128, 16):
            acc_v[rr, pl.ds(o, 16)] = neg

    def pool_table(x_hbm, tbl_hbm, aoff):
        nbase = wid * ROWS_W

        def idx_rd(c, buf):
            return pltpu.async_copy(
                x_hbm.at[pl.ds(nbase + c * GPC, GPC)], buf, semi)

        def idx_wait(c, buf):
            pltpu.make_async_copy(
                x_hbm.at[pl.ds(nbase + c * GPC, GPC)], buf, semi).wait()

        def gat(ibuf, rbuf):
            for j in range(GPC):
                pltpu.async_copy(tbl_hbm.at[ibuf.at[j]], rbuf.at[j], semg)

        def gat_wait(ibuf, rbuf):
            for j in range(GPC):
                pltpu.make_async_copy(tbl_hbm.at[ibuf.at[j]], rbuf.at[j],
                                      semg).wait()

        def accum(rbuf):
            zero = jnp.zeros((16,), jnp.float32)
            for g in range(GPC):

                def hbody(i, acc):
                    a0, a1, a2, a3 = acc
                    for u in range(5):
                        r = i * 5 + u
                        a0 = a0 + rbuf[g, r, pl.ds(0, 16)]
                        a1 = a1 + rbuf[g, r, pl.ds(16, 16)]
                        a2 = a2 + rbuf[g, r, pl.ds(32, 16)]
                        a3 = a3 + rbuf[g, r, pl.ds(34, 16)]
                    return (a0, a1, a2, a3)

                a0, a1, a2, a3 = lax.fori_loop(
                    0, HIST // 5, hbody, (zero, zero, zero, zero))
                for o, a in ((0, a0), (16, a1), (32, a2), (48, a3)):
                    acc_v[0, pl.ds(aoff + o, 16)] = jnp.maximum(
                        acc_v[0, pl.ds(aoff + o, 16)], a)

        npair = NCHUNK // 2
        idx_rd(0, idx_a)
        idx_rd(1, idx_b)
        idx_wait(0, idx_a)
        gat(idx_a, rows_a)

        def pair_body(k, carry):
            c_a = 2 * k
            gat_wait(idx_a, rows_a)
            idx_wait(c_a + 1, idx_b)
            gat(idx_b, rows_b)

            @pl.when(k < npair - 1)
            def _():
                idx_rd(c_a + 2, idx_a)

            accum(rows_a)
            gat_wait(idx_b, rows_b)

            @pl.when(k < npair - 1)
            def _():
                idx_wait(c_a + 2, idx_a)
                gat(idx_a, rows_a)
                idx_rd(c_a + 3, idx_b)

            accum(rows_b)
            return carry

        lax.fori_loop(0, npair, pair_body, 0)

    pool_table(xr_hbm, etd_hbm, 0)
    pool_table(xw_hbm, ewae_hbm, 64)
    pltpu.sync_copy(acc_v, out_hbm.at[pl.ds(wid * 8, 8)])


_sc_pool = pl.kernel(
    _sc_body,
    out_type=jax.ShapeDtypeStruct((NW * 8, 128), jnp.float32),
    mesh=_mesh,
    scratch_types=[
        pltpu.VMEM((GPC, HIST), jnp.int32),
        pltpu.VMEM((GPC, HIST), jnp.int32),
        pltpu.VMEM((GPC, HIST, PDIM), jnp.float32),
        pltpu.VMEM((GPC, HIST, PDIM), jnp.float32),
        pltpu.VMEM((8, 128), jnp.float32),
        pltpu.SemaphoreType.DMA,
        pltpu.SemaphoreType.DMA,
    ],
    compiler_params=pltpu.CompilerParams(use_tc_tiling_on_sc=False),
)


def _head_body(p_ref, w_ref, b_ref, y_ref, pred_ref, loss_ref):
    m = jnp.max(p_ref[...], axis=0, keepdims=True)
    hid = jnp.concatenate(
        [m[:, 0:48], m[:, 62:64], m[:, 64:112], m[:, 126:128]],
        axis=1) * (1.0 / HIST)
    logits = lax.dot_general(
        hid, w_ref[...], (((1,), (1,)), ((), ()))) + b_ref[...]
    ex = jnp.exp(logits - jnp.max(logits, axis=1, keepdims=True))
    pred = ex / jnp.sum(ex, axis=1, keepdims=True)
    mx = jnp.max(pred, axis=1, keepdims=True)
    lse = jnp.log(jnp.sum(jnp.exp(pred - mx), axis=1, keepdims=True)) + mx
    onehot = (y_ref[...] == 1).astype(jnp.float32)
    picked = jnp.sum(pred * onehot, axis=1, keepdims=True)
    pred_ref[...] = pred
    loss_ref[...] = lse - picked


_head = pl.pallas_call(
    _head_body,
    out_shape=[jax.ShapeDtypeStruct((1, 4), jnp.float32),
               jax.ShapeDtypeStruct((1, 1), jnp.float32)],
)


def kernel(x_random, x_response, y, E_td, E_wae, w_cat, b_cat):
    aux_td = jnp.pad(E_td[VTD - VTD % 128:].T, ((0, 0), (0, 128 - VTD % 128)))
    aux_wae = jnp.pad(E_wae[VWAE - VWAE % 128:].T,
                      ((0, 0), (0, 128 - VWAE % 128)))
    otd, owae = _sc_transpose(E_td.T, E_wae.T, aux_td, aux_wae)
    partials = _sc_pool(x_random, x_response,
                        otd.reshape(VTD, PDIM), owae.reshape(VWAE, PDIM))
    pred, loss = _head(partials, w_cat, b_cat.reshape(1, 4), y.reshape(1, 4))
    return pred.reshape(4), loss.reshape(())
